# baseline (speedup 1.0000x reference)
import jax
import jax.numpy as jnp
from jax.experimental import pallas as pl
from jax.experimental.pallas import tpu as pltpu

MAX_SEQ_LEN = 8192
EMBED_DIM = 1024
_NDMA = 8
_ROWS = MAX_SEQ_LEN // _NDMA


def _copy_body(src_ref, out_ref, *sems):
    copies = []
    for k in range(_NDMA):
        copies.append(pltpu.make_async_copy(
            src_ref.at[pl.ds(k * _ROWS, _ROWS)],
            out_ref.at[pl.ds(k * _ROWS, _ROWS)],
            sems[k]))
    for c in copies:
        c.start()
    for c in copies:
        c.wait()


def kernel(seq_len, pos_embedding):
    del seq_len
    return pl.pallas_call(
        _copy_body,
        in_specs=[pl.BlockSpec(memory_space=pl.ANY)],
        out_specs=pl.BlockSpec(memory_space=pl.ANY),
        out_shape=jax.ShapeDtypeStruct((MAX_SEQ_LEN, EMBED_DIM), jnp.float32),
        scratch_shapes=[pltpu.SemaphoreType.DMA] * _NDMA,
    )(pos_embedding)


# speedup vs baseline: 45.7497x; 45.7497x over previous
import jax
import jax.numpy as jnp
from jax.experimental import pallas as pl
from jax.experimental.pallas import tpu as pltpu

MAX_SEQ_LEN = 8192
EMBED_DIM = 1024
_BLK = 1024


def _copy_body(src_ref, out_ref):
    out_ref[...] = src_ref[...]


def kernel(seq_len, pos_embedding):
    del seq_len
    return pl.pallas_call(
        _copy_body,
        grid=(MAX_SEQ_LEN // _BLK,),
        in_specs=[pl.BlockSpec((_BLK, EMBED_DIM), lambda i: (i, 0))],
        out_specs=pl.BlockSpec((_BLK, EMBED_DIM), lambda i: (i, 0)),
        out_shape=jax.ShapeDtypeStruct((MAX_SEQ_LEN, EMBED_DIM), jnp.float32),
    )(pos_embedding)


# speedup vs baseline: 48.7868x; 1.0664x over previous
import jax
import jax.numpy as jnp
from jax.experimental import pallas as pl
from jax.experimental.pallas import tpu as pltpu

MAX_SEQ_LEN = 8192
EMBED_DIM = 1024
_BLK = 2048


def _copy_body(src_ref, out_ref):
    out_ref[...] = src_ref[...]


def kernel(seq_len, pos_embedding):
    del seq_len
    return pl.pallas_call(
        _copy_body,
        grid=(MAX_SEQ_LEN // _BLK,),
        in_specs=[pl.BlockSpec((_BLK, EMBED_DIM), lambda i: (i, 0))],
        out_specs=pl.BlockSpec((_BLK, EMBED_DIM), lambda i: (i, 0)),
        out_shape=jax.ShapeDtypeStruct((MAX_SEQ_LEN, EMBED_DIM), jnp.float32),
    )(pos_embedding)
